# Initial kernel scaffold; baseline (speedup 1.0000x reference)
import jax
import jax.numpy as jnp
from jax.experimental import pallas as pl


def kernel(cat_input, non_cat_input, tables, W1, b1, W2, b2):
    raise NotImplementedError("write your pallas kernel here")


# speedup vs baseline: 7.4697x; 7.4697x over previous
import functools

import jax
import jax.numpy as jnp
from jax import lax
from jax.experimental import pallas as pl
from jax.experimental.pallas import tpu as pltpu
from jax.experimental.pallas import tpu_sc as plsc

B = 16384
NF = 26
V = 100000
E = 32
D_DENSE = 13
H = 1028

NW = 32
IDX_W = B * NF // NW
CHUNK = 128
NCHUNK = IDX_W // CHUNK


def _gather_body(tbl_hbm, idx_hbm, out_hbm, idx_v, rows_v, gsem, osem):
    wid = lax.axis_index("s") * 2 + lax.axis_index("c")
    pltpu.sync_copy(idx_hbm.at[pl.ds(wid * NCHUNK, NCHUNK)], idx_v)

    def step(c, _):
        slot = lax.rem(c, 2)
        row_base = (wid * NCHUNK + c) * CHUNK
        pltpu.async_copy(tbl_hbm.at[idx_v.at[c]], rows_v.at[slot], gsem).wait()
        pltpu.async_copy(rows_v.at[slot], out_hbm.at[pl.ds(row_base, CHUNK)],
                         osem).wait()
        return 0

    lax.fori_loop(0, NCHUNK, step, 0)


@functools.partial(
    pl.kernel,
    out_type=jax.ShapeDtypeStruct((B * NF, E), jnp.float32),
    mesh=plsc.VectorSubcoreMesh(core_axis_name="c", subcore_axis_name="s"),
    compiler_params=pltpu.CompilerParams(use_tc_tiling_on_sc=False),
    scratch_types=[
        pltpu.VMEM((NCHUNK, CHUNK), jnp.int32),
        pltpu.VMEM((2, CHUNK, E), jnp.float32),
        pltpu.SemaphoreType.DMA,
        pltpu.SemaphoreType.DMA,
    ],
)
def _sc_gather(tbl_hbm, idx_hbm, out_hbm, idx_v, rows_v, gsem, osem):
    _gather_body(tbl_hbm, idx_hbm, out_hbm, idx_v, rows_v, gsem, osem)


def _mlp_body(x_ref, nc_ref, w1e_ref, w1d_ref, b1_ref, w2_ref, o_ref):
    acc = jnp.dot(x_ref[...], w1e_ref[...], preferred_element_type=jnp.float32)
    acc = acc + jnp.dot(nc_ref[...], w1d_ref[...],
                        preferred_element_type=jnp.float32)
    acc = acc + b1_ref[...]
    h = jnp.maximum(acc, 0.0)
    o_ref[...] = jnp.dot(h, w2_ref[...], preferred_element_type=jnp.float32)


def _mlp(x_emb, non_cat, w1e, w1d, b1, w2):
    BM = 1024
    grid = (B // BM,)
    return pl.pallas_call(
        _mlp_body,
        grid=grid,
        in_specs=[
            pl.BlockSpec((BM, NF * E), lambda i: (i, 0)),
            pl.BlockSpec((BM, D_DENSE), lambda i: (i, 0)),
            pl.BlockSpec((NF * E, H), lambda i: (0, 0)),
            pl.BlockSpec((D_DENSE, H), lambda i: (0, 0)),
            pl.BlockSpec((1, H), lambda i: (0, 0)),
            pl.BlockSpec((H, 1), lambda i: (0, 0)),
        ],
        out_specs=pl.BlockSpec((BM, 1), lambda i: (i, 0)),
        out_shape=jax.ShapeDtypeStruct((B, 1), jnp.float32),
    )(x_emb, non_cat, w1e, w1d, b1, w2)


def kernel(cat_input, non_cat_input, tables, W1, b1, W2, b2):
    tbl_flat = tables.reshape(NF * V, E)
    idx_flat = (cat_input + (jnp.arange(NF, dtype=jnp.int32) * V)[None, :])
    idx_flat = idx_flat.reshape(NW * NCHUNK, CHUNK)
    rows = _sc_gather(tbl_flat, idx_flat)
    x_emb = rows.reshape(B, NF * E)
    out = _mlp(x_emb, non_cat_input, W1[:NF * E], W1[NF * E:],
               b1.reshape(1, H), W2)
    return out.reshape(-1) + b2[0]


# speedup vs baseline: 7.4754x; 1.0008x over previous
import functools

import jax
import jax.numpy as jnp
from jax import lax
from jax.experimental import pallas as pl
from jax.experimental.pallas import tpu as pltpu
from jax.experimental.pallas import tpu_sc as plsc

B = 16384
NF = 26
V = 100000
E = 32
D_DENSE = 13
H = 1028

NW = 32
IDX_W = B * NF // NW
CHUNK = 128
NCHUNK = IDX_W // CHUNK


def _gather_body(tbl_hbm, idx_hbm, out_hbm, idx_v, rows_v, gsem, osem):
    wid = lax.axis_index("s") * 2 + lax.axis_index("c")
    pltpu.sync_copy(idx_hbm.at[pl.ds(wid * NCHUNK, NCHUNK)], idx_v)

    def step(c, _):
        slot = lax.rem(c, 2)
        row_base = (wid * NCHUNK + c) * CHUNK
        pltpu.async_copy(tbl_hbm.at[idx_v.at[c]], rows_v.at[slot], gsem).wait()
        pltpu.async_copy(rows_v.at[slot], out_hbm.at[pl.ds(row_base, CHUNK)],
                         osem).wait()
        return 0

    lax.fori_loop(0, NCHUNK, step, 0)


@functools.partial(
    pl.kernel,
    out_type=jax.ShapeDtypeStruct((B * NF, E), jnp.float32),
    mesh=plsc.VectorSubcoreMesh(core_axis_name="c", subcore_axis_name="s"),
    compiler_params=pltpu.CompilerParams(use_tc_tiling_on_sc=False),
    scratch_types=[
        pltpu.VMEM((NCHUNK, CHUNK), jnp.int32),
        pltpu.VMEM((2, CHUNK, E), jnp.float32),
        pltpu.SemaphoreType.DMA,
        pltpu.SemaphoreType.DMA,
    ],
)
def _sc_gather(tbl_hbm, idx_hbm, out_hbm, idx_v, rows_v, gsem, osem):
    _gather_body(tbl_hbm, idx_hbm, out_hbm, idx_v, rows_v, gsem, osem)


def _mlp_body(x_ref, nc_ref, w1e_ref, w1d_ref, b1_ref, w2_ref, o_ref):
    x = x_ref[...].astype(jnp.bfloat16)
    acc = jnp.dot(x, w1e_ref[...], preferred_element_type=jnp.float32)
    acc = acc + jnp.dot(nc_ref[...], w1d_ref[...],
                        preferred_element_type=jnp.float32)
    acc = acc + b1_ref[...]
    h = jnp.maximum(acc, 0.0).astype(jnp.bfloat16)
    o_ref[...] = jnp.dot(h, w2_ref[...], preferred_element_type=jnp.float32)


def _mlp(x_emb, non_cat, w1e, w1d, b1, w2):
    BM = 1024
    grid = (B // BM,)
    return pl.pallas_call(
        _mlp_body,
        grid=grid,
        in_specs=[
            pl.BlockSpec((BM, NF * E), lambda i: (i, 0)),
            pl.BlockSpec((BM, D_DENSE), lambda i: (i, 0)),
            pl.BlockSpec((NF * E, H), lambda i: (0, 0)),
            pl.BlockSpec((D_DENSE, H), lambda i: (0, 0)),
            pl.BlockSpec((1, H), lambda i: (0, 0)),
            pl.BlockSpec((H, 1), lambda i: (0, 0)),
        ],
        out_specs=pl.BlockSpec((BM, 1), lambda i: (i, 0)),
        out_shape=jax.ShapeDtypeStruct((B, 1), jnp.float32),
    )(x_emb, non_cat, w1e, w1d, b1, w2)


def kernel(cat_input, non_cat_input, tables, W1, b1, W2, b2):
    tbl_flat = tables.reshape(NF * V, E)
    idx_flat = (cat_input + (jnp.arange(NF, dtype=jnp.int32) * V)[None, :])
    idx_flat = idx_flat.reshape(NW * NCHUNK, CHUNK)
    rows = _sc_gather(tbl_flat, idx_flat)
    x_emb = rows.reshape(B, NF * E)
    out = _mlp(x_emb, non_cat_input.astype(jnp.bfloat16),
               W1[:NF * E].astype(jnp.bfloat16),
               W1[NF * E:].astype(jnp.bfloat16),
               b1.reshape(1, H), W2.astype(jnp.bfloat16))
    return out.reshape(-1) + b2[0]
